# Initial kernel scaffold; baseline (speedup 1.0000x reference)
import jax
import jax.numpy as jnp
from jax.experimental import pallas as pl


def kernel(fips_idx, crop_idx, year_idx, growth_stage, fips_table, crop_table, year_table, W1, b1, W2, b2):
    raise NotImplementedError("write your pallas kernel here")


# speedup vs baseline: 1.7274x; 1.7274x over previous
import functools

import jax
import jax.numpy as jnp
from jax import lax
from jax.experimental import pallas as pl
from jax.experimental.pallas import tpu as pltpu
from jax.experimental.pallas import tpu_sc as plsc

_BATCH = 16384
_EMB = 32
_OUT = 64
_NCROP = 4
_NYEAR = 6

_NC, _NS = 2, 16
_NW = _NC * _NS
_BPW = _BATCH // _NW

_BB = 2048


def _sc_gather(table4, idx):
    mesh = plsc.VectorSubcoreMesh(core_axis_name="c", subcore_axis_name="s")

    @functools.partial(
        pl.kernel,
        mesh=mesh,
        out_type=jax.ShapeDtypeStruct((_BATCH, 4 * _EMB), jnp.float32),
        scratch_types=[
            pltpu.VMEM((_BPW,), jnp.int32),
            pltpu.VMEM((_BPW, 4 * _EMB), jnp.float32),
            pltpu.SemaphoreType.DMA,
        ],
    )
    def k(table_hbm, idx_hbm, out_hbm, idx_v, rows_v, sem):
        wid = lax.axis_index("s") * _NC + lax.axis_index("c")
        base = wid * _BPW
        pltpu.sync_copy(idx_hbm.at[pl.ds(base, _BPW)], idx_v)

        @pl.loop(0, _BPW, step=16)
        def _(c):
            idx_v[pl.ds(c, 16)] = lax.shift_right_logical(idx_v[pl.ds(c, 16)], 2)

        pltpu.async_copy(table_hbm.at[idx_v], rows_v, sem).wait()
        pltpu.sync_copy(rows_v, out_hbm.at[pl.ds(base, _BPW)])

    return k(table4, idx)


def _mlp_body(rows_ref, fi_ref, ci_ref, yi_ref, gs_ref, ct_ref, yt_ref,
              w1_ref, b1_ref, w2_ref, b2_ref, o_ref):
    rows = rows_ref[...]
    fi = fi_ref[...]
    ci = ci_ref[...]
    yi = yi_ref[...]
    gs = gs_ref[...]
    w1 = w1_ref[...]

    lo = jnp.bitwise_and(fi, 3)
    fe = jnp.zeros((rows.shape[0], _EMB), jnp.float32)
    for kk in range(4):
        mask = (lo == kk).astype(jnp.float32)
        fe += mask * rows[:, kk * _EMB:(kk + 1) * _EMB]

    crop_oh = (lax.broadcasted_iota(jnp.int32, (rows.shape[0], _NCROP), 1)
               == ci).astype(jnp.float32)
    year_oh = (lax.broadcasted_iota(jnp.int32, (rows.shape[0], _NYEAR), 1)
               == yi).astype(jnp.float32)
    ce = jnp.dot(crop_oh, ct_ref[...], preferred_element_type=jnp.float32)
    ye = jnp.dot(year_oh, yt_ref[...], preferred_element_type=jnp.float32)

    h = jnp.dot(fe, w1[0:_EMB], preferred_element_type=jnp.float32)
    h += jnp.dot(ce, w1[_EMB:2 * _EMB], preferred_element_type=jnp.float32)
    h += jnp.dot(ye, w1[2 * _EMB:3 * _EMB], preferred_element_type=jnp.float32)
    h += gs * w1[3 * _EMB:3 * _EMB + 1]
    h = jnp.maximum(h + b1_ref[...], 0.0)
    h = jnp.dot(h, w2_ref[...], preferred_element_type=jnp.float32)
    h = jnp.maximum(h + b2_ref[...], 0.0)
    o_ref[...] = h


def _tc_mlp(rows, fi, ci, yi, gs, ct, yt, w1, b1, w2, b2):
    grid = (_BATCH // _BB,)
    batch_spec = lambda cols: pl.BlockSpec((_BB, cols), lambda i: (i, 0))
    const_spec = lambda shape: pl.BlockSpec(shape, lambda i: (0, 0))
    return pl.pallas_call(
        _mlp_body,
        grid=grid,
        in_specs=[
            batch_spec(4 * _EMB),
            batch_spec(1),
            batch_spec(1),
            batch_spec(1),
            batch_spec(1),
            const_spec((_NCROP, _EMB)),
            const_spec((_NYEAR, _EMB)),
            const_spec((3 * _EMB + 1, _OUT)),
            const_spec((1, _OUT)),
            const_spec((_OUT, _OUT)),
            const_spec((1, _OUT)),
        ],
        out_specs=batch_spec(_OUT),
        out_shape=jax.ShapeDtypeStruct((_BATCH, _OUT), jnp.float32),
    )(rows, fi, ci, yi, gs, ct, yt, w1, b1, w2, b2)


def kernel(fips_idx, crop_idx, year_idx, growth_stage, fips_table, crop_table,
           year_table, W1, b1, W2, b2):
    fi = fips_idx.astype(jnp.int32)
    table4 = fips_table.reshape(25000, 4 * _EMB)
    rows = _sc_gather(table4, fi)
    return _tc_mlp(
        rows,
        fi.reshape(_BATCH, 1),
        crop_idx.reshape(_BATCH, 1),
        year_idx.reshape(_BATCH, 1),
        growth_stage.reshape(_BATCH, 1),
        crop_table,
        year_table,
        W1,
        b1.reshape(1, _OUT),
        W2,
        b2.reshape(1, _OUT),
    )
